# baseline (speedup 1.0000x reference)
import functools

import jax
import jax.numpy as jnp
from jax import lax
from jax.experimental import pallas as pl
from jax.experimental.pallas import tpu as pltpu
from jax.experimental.pallas import tpu_sc as plsc

B, P, S, L = 1024, 16384, 8, 32
NC, NS = 2, 16
NW = NC * NS
LANES = 16
GROWS = P * 4 // 128
G = 4
SLABW = 2048
NSLAB = P // SLABW
YB = SLABW // 128
RB = 4
MB = B // RB // G


def _sc_body(params_hbm, y_hbm, a2s_hbm, pre_hbm, add_hbm, del_hbm,
             a2s_v, pidx_v, plane_v, y0_v, y1_v, sv0_v, sv1_v,
             o00, o01, o02, o10, o11, o12,
             ysem0, ysem1, gsem, osem0, osem1):
    wid = lax.axis_index("s") * NC + lax.axis_index("c")
    slab = wid // G
    par = wid % G
    col0 = slab * SLABW

    zf = jnp.zeros((LANES,), jnp.float32)
    iota = lax.iota(jnp.int32, LANES)
    sel0 = jnp.zeros((LANES,), jnp.int32)

    obs = ((o00, o01, o02), (o10, o11, o12))
    yvs = (y0_v, y1_v)
    svs = (sv0_v, sv1_v)
    ysems = (ysem0, ysem1)
    osems = (osem0, osem1)
    outs = (pre_hbm, add_hbm, del_hbm)

    pltpu.sync_copy(a2s_hbm, a2s_v)
    for m in range(S * YB * 3 // LANES):
        j16 = iota + m * LANES
        s = j16 // (YB * 3)
        rem = j16 - s * (YB * 3)
        yb = rem // 3
        cp = rem - yb * 3
        pidx_v[pl.ds(m * LANES, LANES)] = (s * 128 + slab * YB + yb) * 4 + cp + 1
    qtr = S * YB * 3 // 4
    gcps = [pltpu.async_copy(params_hbm.at[pidx_v.at[pl.ds(q * qtr, qtr)]],
                             plane_v.at[pl.ds(q * qtr, qtr)], gsem)
            for q in range(4)]

    def _zero(i, _):
        row = i >> (SLABW // LANES).bit_length() - 1
        off = (i & (SLABW // LANES - 1)) * LANES
        for bset in obs:
            for bref in bset:
                bref[row, pl.ds(off, LANES)] = zf
        return 0

    lax.fori_loop(0, RB * SLABW // LANES, _zero, 0)
    for gcp in gcps:
        gcp.wait()

    pltpu.async_copy(y_hbm.at[pl.ds(par * RB, RB)], y0_v, ysem0)

    def _pair(g, _):
        for k in (0, 1):
            m = g * 2 + k
            b0 = (G * m + par) * RB
            pltpu.make_async_copy(y_hbm.at[pl.ds(b0, RB)], yvs[k],
                                  ysems[k]).wait()
            nxt = jnp.where(b0 + G * RB >= B, 0, b0 + G * RB)
            pltpu.async_copy(y_hbm.at[pl.ds(nxt, RB)], yvs[1 - k],
                             ysems[1 - k])

            @pl.when(g > 0)
            def _():
                for bref in obs[k]:
                    pltpu.make_async_copy(
                        bref,
                        pre_hbm.at[pl.ds(b0 - 2 * G * RB, RB),
                                   pl.ds(col0, SLABW)],
                        osems[k]).wait()

                def _rz(r, _):
                    rv = sel0 + r
                    for h in (0, 1):
                        sx = plsc.load_gather(svs[k], [rv, iota + h * LANES])
                        mask = sx < SLABW
                        sxc = jnp.minimum(sx, SLABW - 1)
                        for bref in obs[k]:
                            plsc.store_scatter(bref, [rv, sxc], zf, mask=mask)
                    return 0

                lax.fori_loop(0, RB, _rz, 0)

            def _row(r, _):
                rv = sel0 + r
                sbv = plsc.load_gather(a2s_v, [sel0 + b0 + r])
                for h in (0, 1):
                    yh = plsc.load_gather(yvs[k], [rv, iota + h * LANES])
                    mask = (yh >> 11) == slab
                    yl = yh & (SLABW - 1)
                    lane = yh & 127
                    base = (sbv * YB + (yl >> 7)) * 3
                    c1 = plsc.load_gather(plane_v, [base, lane])
                    c2 = plsc.load_gather(plane_v, [base + 1, lane])
                    c3 = plsc.load_gather(plane_v, [base + 2, lane])
                    plsc.addupdate_scatter(obs[k][0], [rv, yl], c2 + c3,
                                           mask=mask)
                    plsc.addupdate_scatter(obs[k][1], [rv, yl], c1, mask=mask)
                    plsc.addupdate_scatter(obs[k][2], [rv, yl], c3, mask=mask)
                    plsc.store_scatter(svs[k], [rv, iota + h * LANES],
                                       jnp.where(mask, yl, SLABW))
                return 0

            lax.fori_loop(0, RB, _row, 0)
            for bref, o in zip(obs[k], outs):
                pltpu.async_copy(
                    bref, o.at[pl.ds(b0, RB), pl.ds(col0, SLABW)], osems[k])
        return 0

    lax.fori_loop(0, MB // 2, _pair, 0)

    for k in (0, 1):
        b0 = (G * (MB - 2 + k) + par) * RB
        for bref, o in zip(obs[k], outs):
            pltpu.make_async_copy(
                bref, o.at[pl.ds(b0, RB), pl.ds(col0, SLABW)],
                osems[k]).wait()
    pltpu.make_async_copy(y_hbm.at[pl.ds(0, RB)], yvs[0], ysems[0]).wait()


@functools.partial(jax.jit, donate_argnums=())
def kernel(schema_params, y_indices, action_to_schema):
    params2d = (schema_params
                .reshape(S, P // 128, 128, 4)
                .transpose(0, 1, 3, 2)
                .reshape(GROWS * S, 128))
    mesh = plsc.VectorSubcoreMesh(core_axis_name="c", subcore_axis_name="s")
    out = jax.ShapeDtypeStruct((B, P), jnp.float32)
    run = pl.kernel(
        _sc_body,
        out_type=[out, out, out],
        mesh=mesh,
        compiler_params=pltpu.CompilerParams(needs_layout_passes=False),
        scratch_types=[
            pltpu.VMEM((B,), jnp.int32),
            pltpu.VMEM((S * YB * 3,), jnp.int32),
            pltpu.VMEM((S * YB * 3, 128), jnp.float32),
            pltpu.VMEM((RB, L), jnp.int32),
            pltpu.VMEM((RB, L), jnp.int32),
            pltpu.VMEM((RB, L), jnp.int32),
            pltpu.VMEM((RB, L), jnp.int32),
            pltpu.VMEM((RB, SLABW), jnp.float32),
            pltpu.VMEM((RB, SLABW), jnp.float32),
            pltpu.VMEM((RB, SLABW), jnp.float32),
            pltpu.VMEM((RB, SLABW), jnp.float32),
            pltpu.VMEM((RB, SLABW), jnp.float32),
            pltpu.VMEM((RB, SLABW), jnp.float32),
            pltpu.SemaphoreType.DMA,
            pltpu.SemaphoreType.DMA,
            pltpu.SemaphoreType.DMA,
            pltpu.SemaphoreType.DMA,
            pltpu.SemaphoreType.DMA,
        ],
    )
    pre, add, dele = run(params2d, y_indices, action_to_schema)
    return (pre, add, dele)
